# Initial kernel scaffold; baseline (speedup 1.0000x reference)
import functools

import jax
import jax.numpy as jnp
from jax import lax
from jax.experimental import pallas as pl
from jax.experimental.pallas import tpu as pltpu
from jax.experimental.pallas import tpu_sc as plsc

N_NODES = 100000
N_EDGES = 1600000
CIN = 4

NC = 2
NS = 16
NW = NC * NS
NP = 102400
S = NP // NS

B = 128
SUP = 10
SB = B * SUP
NSUP = N_EDGES // SB

_mesh = plsc.VectorSubcoreMesh(core_axis_name="c", subcore_axis_name="s")


def _worker_id():
    return lax.axis_index("s") * NC + lax.axis_index("c")


@functools.partial(
    pl.kernel,
    out_type=(
        jax.ShapeDtypeStruct((NC, NP), jnp.float32),
        jax.ShapeDtypeStruct((NC, NP), jnp.float32),
    ),
    mesh=_mesh,
    scratch_types=[
        pltpu.VMEM_SHARED((NP,), jnp.float32),
        pltpu.VMEM_SHARED((NP,), jnp.float32),
        pltpu.VMEM((SUP, B), jnp.int32),
        pltpu.VMEM((SUP, B), jnp.int32),
        pltpu.VMEM((SUP, B), jnp.float32),
        pltpu.VMEM((S,), jnp.float32),
    ],
)
def _deg_kernel(ei_ref, w_ref, dego_out, degi_out,
                dego_sh, degi_sh, sidx_v, didx_v, w_v, buf_v):
    cid = lax.axis_index("c")
    sid = lax.axis_index("s")
    wid = _worker_id()
    sl = pl.ds(sid * S, S)

    @pl.loop(0, S // 16)
    def _zero(i):
        buf_v[pl.ds(i * 16, 16)] = jnp.zeros((16,), jnp.float32)

    pltpu.sync_copy(buf_v, dego_sh.at[sl])
    pltpu.sync_copy(buf_v, degi_sh.at[sl])
    plsc.subcore_barrier()

    @pl.loop(wid, NSUP, step=NW)
    def _edges(sc):
        pltpu.sync_copy(ei_ref.at[0, sc], sidx_v)
        pltpu.sync_copy(ei_ref.at[1, sc], didx_v)
        pltpu.sync_copy(w_ref.at[sc], w_v)
        for j in range(SUP):
            pltpu.sync_copy(w_v.at[j], dego_sh.at[sidx_v.at[j]], add=True)
            pltpu.sync_copy(w_v.at[j], degi_sh.at[didx_v.at[j]], add=True)

    plsc.subcore_barrier()
    pltpu.sync_copy(dego_sh.at[sl], buf_v)
    pltpu.sync_copy(buf_v, dego_out.at[cid, sl])
    pltpu.sync_copy(degi_sh.at[sl], buf_v)
    pltpu.sync_copy(buf_v, degi_out.at[cid, sl])


@functools.partial(
    pl.kernel,
    out_type=(
        jax.ShapeDtypeStruct((NC, NP, CIN), jnp.float32),
        jax.ShapeDtypeStruct((NC, NP, CIN), jnp.float32),
    ),
    mesh=_mesh,
    scratch_types=[
        pltpu.VMEM_SHARED((NP, CIN), jnp.float32),
        pltpu.VMEM_SHARED((NP,), jnp.float32),
        pltpu.VMEM_SHARED((NP,), jnp.float32),
        pltpu.VMEM_SHARED((NP, CIN), jnp.float32),
        pltpu.VMEM_SHARED((NP, CIN), jnp.float32),
        pltpu.VMEM((S,), jnp.float32),
        pltpu.VMEM((S,), jnp.float32),
        pltpu.VMEM((S,), jnp.float32),
        pltpu.VMEM((SUP, B), jnp.int32),
        pltpu.VMEM((SUP, B), jnp.int32),
        pltpu.VMEM((SUP, B), jnp.float32),
        pltpu.VMEM((B,), jnp.float32),
        pltpu.VMEM((B,), jnp.float32),
        pltpu.VMEM((B,), jnp.float32),
        pltpu.VMEM((B,), jnp.float32),
        pltpu.VMEM((B, CIN), jnp.float32),
        pltpu.VMEM((B, CIN), jnp.float32),
        pltpu.VMEM((B, CIN), jnp.float32),
        pltpu.VMEM((B, CIN), jnp.float32),
    ],
)
def _scatter_kernel(ei_ref, w_ref, x_ref, degop_ref, degip_ref, zeros4_ref,
                    ao_out, ai_out,
                    x_sh, rdo_sh, rdi_sh, ao_sh, ai_sh,
                    d0_v, d1_v, r_v, sidx_v, didx_v, w_v,
                    rdo_v, rdi_v, no_v, ni_v, xs_v, xd_v, co_v, ci_v):
    cid = lax.axis_index("c")
    sid = lax.axis_index("s")
    wid = _worker_id()
    sl = pl.ds(sid * S, S)

    pltpu.sync_copy(x_ref.at[sl, :], x_sh.at[sl, :])
    pltpu.sync_copy(zeros4_ref.at[sl, :], ao_sh.at[sl, :])
    pltpu.sync_copy(zeros4_ref.at[sl, :], ai_sh.at[sl, :])

    pltpu.sync_copy(degop_ref.at[0, sl], d0_v)
    pltpu.sync_copy(degop_ref.at[1, sl], d1_v)

    @pl.loop(0, S // 16)
    def _rdo(i):
        ds16 = pl.ds(i * 16, 16)
        r_v[ds16] = 1.0 / (d0_v[ds16] + d1_v[ds16])

    pltpu.sync_copy(r_v, rdo_sh.at[sl])
    pltpu.sync_copy(degip_ref.at[0, sl], d0_v)
    pltpu.sync_copy(degip_ref.at[1, sl], d1_v)

    @pl.loop(0, S // 16)
    def _rdi(i):
        ds16 = pl.ds(i * 16, 16)
        r_v[ds16] = 1.0 / (d0_v[ds16] + d1_v[ds16])

    pltpu.sync_copy(r_v, rdi_sh.at[sl])
    plsc.subcore_barrier()

    lane = lax.iota(jnp.int32, 16)
    erep = lane >> 2
    crep = lane & 3

    @pl.loop(wid, NSUP, step=NW)
    def _edges(sc):
        pltpu.sync_copy(ei_ref.at[0, sc], sidx_v)
        pltpu.sync_copy(ei_ref.at[1, sc], didx_v)
        pltpu.sync_copy(w_ref.at[sc], w_v)
        for j in range(SUP):
            si = sidx_v.at[j]
            di = didx_v.at[j]
            pltpu.sync_copy(rdo_sh.at[si], rdo_v)
            pltpu.sync_copy(rdi_sh.at[di], rdi_v)
            pltpu.sync_copy(x_sh.at[si], xs_v)
            pltpu.sync_copy(x_sh.at[di], xd_v)

            @pl.loop(0, B // 16)
            def _norm(i):
                ds16 = pl.ds(i * 16, 16)
                wv = w_v[j, ds16]
                no_v[ds16] = wv * rdo_v[ds16]
                ni_v[ds16] = wv * rdi_v[ds16]

            @pl.loop(0, B * CIN // 16)
            def _mul(i):
                e16 = i * 4 + erep
                ng = plsc.load_gather(no_v, [e16])
                xg = plsc.load_gather(xs_v, [e16, crep])
                plsc.store_scatter(co_v, [e16, crep], xg * ng)
                ng2 = plsc.load_gather(ni_v, [e16])
                xg2 = plsc.load_gather(xd_v, [e16, crep])
                plsc.store_scatter(ci_v, [e16, crep], xg2 * ng2)

            pltpu.sync_copy(co_v, ao_sh.at[di], add=True)
            pltpu.sync_copy(ci_v, ai_sh.at[si], add=True)

    plsc.subcore_barrier()
    pltpu.sync_copy(ao_sh.at[sl, :], ao_out.at[cid, sl, :])
    pltpu.sync_copy(ai_sh.at[sl, :], ai_out.at[cid, sl, :])


BK = 2048


def _dense_body(xT_ref, aoT_ref, aiT_ref, wz_ref, bz_ref, wh_ref, bh_ref,
                wl_ref, bl_ref, out_ref):
    xb = xT_ref[...]
    ao = aoT_ref[0] + aoT_ref[1]
    ai = aiT_ref[0] + aiT_ref[1]
    f = jnp.concatenate([xb, ao, ai], axis=0)
    pz = jnp.dot(wz_ref[...], f, preferred_element_type=jnp.float32) + bz_ref[...]
    ph = jnp.dot(wh_ref[...], f, preferred_element_type=jnp.float32) + bh_ref[...]
    z = jax.nn.sigmoid(pz)
    ht = jnp.tanh(ph)
    g = jnp.maximum((1.0 - z) * ht, 0.0)
    out_ref[...] = jnp.dot(wl_ref[...], g, preferred_element_type=jnp.float32) + bl_ref[...]


_dense_call = pl.pallas_call(
    _dense_body,
    grid=(NP // BK,),
    in_specs=[
        pl.BlockSpec((4, BK), lambda i: (0, i)),
        pl.BlockSpec((NC, 4, BK), lambda i: (0, 0, i)),
        pl.BlockSpec((NC, 4, BK), lambda i: (0, 0, i)),
        pl.BlockSpec((32, 12), lambda i: (0, 0)),
        pl.BlockSpec((32, 1), lambda i: (0, 0)),
        pl.BlockSpec((32, 12), lambda i: (0, 0)),
        pl.BlockSpec((32, 1), lambda i: (0, 0)),
        pl.BlockSpec((1, 32), lambda i: (0, 0)),
        pl.BlockSpec((1, 1), lambda i: (0, 0)),
    ],
    out_specs=pl.BlockSpec((1, BK), lambda i: (0, i)),
    out_shape=jax.ShapeDtypeStruct((1, NP), jnp.float32),
)


def kernel(x, edge_index, edge_weight, W_z, b_z, W_r, b_r, W_h, b_h,
           W_lin, b_lin):
    del W_r, b_r
    ei2 = edge_index.reshape(2, NSUP, SUP, B)
    w2 = edge_weight.reshape(NSUP, SUP, B)
    x_pad = jnp.pad(x, ((0, NP - N_NODES), (0, 0)))
    zeros4 = jnp.zeros((NP, CIN), jnp.float32)

    dego_p, degi_p = _deg_kernel(ei2, w2)
    ao_p, ai_p = _scatter_kernel(ei2, w2, x_pad, dego_p, degi_p, zeros4)

    wz_f = jnp.concatenate(
        [W_z[0, 0, :CIN] + W_z[1, 0, :CIN], W_z[0, 1, :CIN], W_z[1, 1, :CIN]],
        axis=0).T
    wh_f = jnp.concatenate(
        [W_h[0, 0, :CIN] + W_h[1, 0, :CIN], W_h[0, 1, :CIN], W_h[1, 1, :CIN]],
        axis=0).T

    out_t = _dense_call(
        x_pad.T,
        jnp.transpose(ao_p, (0, 2, 1)),
        jnp.transpose(ai_p, (0, 2, 1)),
        wz_f, b_z[:, None], wh_f, b_h[:, None],
        W_lin.T, b_lin[:, None],
    )
    return out_t[0, :N_NODES][:, None]


# speedup vs baseline: 47.2997x; 47.2997x over previous
import functools

import jax
import jax.numpy as jnp
from jax import lax
from jax.experimental import pallas as pl
from jax.experimental.pallas import tpu as pltpu
from jax.experimental.pallas import tpu_sc as plsc

N_NODES = 100000
N_EDGES = 1600000
CIN = 4

NC = 2
NS = 16
NW = NC * NS
NP = 102400
S = NP // NS
SBLK = 1600

B = 128
SUP = 16
SB = B * SUP
NSUP = -(-N_EDGES // SB)
EP = NSUP * SB

_mesh = plsc.VectorSubcoreMesh(core_axis_name="c", subcore_axis_name="s")
_sc_params = pltpu.CompilerParams(
    needs_layout_passes=False, use_tc_tiling_on_sc=False)


@functools.partial(
    pl.kernel,
    out_type=(
        jax.ShapeDtypeStruct((NC * NP,), jnp.float32),
        jax.ShapeDtypeStruct((NC * NP,), jnp.float32),
    ),
    mesh=_mesh,
    compiler_params=_sc_params,
    scratch_types=[
        pltpu.VMEM_SHARED((NP,), jnp.float32),
        pltpu.VMEM_SHARED((NP,), jnp.float32),
        pltpu.VMEM((SUP, B), jnp.int32),
        pltpu.VMEM((SUP, B), jnp.int32),
        pltpu.VMEM((SUP, B), jnp.float32),
        pltpu.VMEM((S,), jnp.float32),
        pltpu.SemaphoreType.DMA,
    ],
)
def _deg_kernel(ei_ref, w_ref, dego_out, degi_out,
                dego_sh, degi_sh, sidx_v, didx_v, w_v, buf_v, ssem):
    cid = lax.axis_index("c")
    sid = lax.axis_index("s")
    wid = sid * NC + cid
    sl = pl.ds(sid * S, S)

    @pl.loop(0, S // 16)
    def _zero(i):
        buf_v[pl.ds(i * 16, 16)] = jnp.zeros((16,), jnp.float32)

    pltpu.sync_copy(buf_v, dego_sh.at[sl])
    pltpu.sync_copy(buf_v, degi_sh.at[sl])
    plsc.subcore_barrier()

    @pl.loop(wid, NSUP, step=NW)
    def _edges(sc):
        pltpu.sync_copy(ei_ref.at[0, sc], sidx_v)
        pltpu.sync_copy(ei_ref.at[1, sc], didx_v)
        pltpu.sync_copy(w_ref.at[sc], w_v)
        for j in range(SUP):
            pltpu.sync_copy(w_v.at[j], dego_sh.at[sidx_v.at[j]], add=True)
            pltpu.sync_copy(w_v.at[j], degi_sh.at[didx_v.at[j]], add=True)

    plsc.subcore_barrier()
    osl = pl.ds(cid * NP + sid * S, S)
    pltpu.sync_copy(dego_sh.at[sl], buf_v)
    pltpu.sync_copy(buf_v, dego_out.at[osl])
    pltpu.sync_copy(degi_sh.at[sl], buf_v)
    pltpu.sync_copy(buf_v, degi_out.at[osl])


@functools.partial(
    pl.kernel,
    out_type=(
        jax.ShapeDtypeStruct((NC * NP, CIN), jnp.float32),
        jax.ShapeDtypeStruct((NC * NP, CIN), jnp.float32),
    ),
    mesh=_mesh,
    compiler_params=_sc_params,
    scratch_types=[
        pltpu.VMEM_SHARED((NP, CIN), jnp.float32),
        pltpu.VMEM_SHARED((NP,), jnp.float32),
        pltpu.VMEM_SHARED((NP,), jnp.float32),
        pltpu.VMEM_SHARED((NP, CIN), jnp.float32),
        pltpu.VMEM_SHARED((NP, CIN), jnp.float32),
        pltpu.VMEM((SBLK,), jnp.float32),
        pltpu.VMEM((SBLK,), jnp.float32),
        pltpu.VMEM((SBLK,), jnp.float32),
        pltpu.VMEM((SUP, B), jnp.int32),
        pltpu.VMEM((SUP, B), jnp.int32),
        pltpu.VMEM((SUP, B), jnp.float32),
        pltpu.VMEM((B,), jnp.float32),
        pltpu.VMEM((B,), jnp.float32),
        pltpu.VMEM((B,), jnp.float32),
        pltpu.VMEM((B,), jnp.float32),
        pltpu.VMEM((B, CIN), jnp.float32),
        pltpu.VMEM((B, CIN), jnp.float32),
        pltpu.VMEM((B, CIN), jnp.float32),
        pltpu.VMEM((B, CIN), jnp.float32),
        pltpu.SemaphoreType.DMA,
        pltpu.SemaphoreType.DMA,
    ],
)
def _scatter_kernel(ei_ref, w_ref, x_ref, degop_ref, degip_ref, zeros4_ref,
                    ao_out, ai_out,
                    x_sh, rdo_sh, rdi_sh, ao_sh, ai_sh,
                    d0_v, d1_v, r_v, sidx_v, didx_v, w_v,
                    rdo_b, rdi_b, no_v, ni_v, xs_b, xd_b, co_b, ci_b,
                    gsem, ssem):
    cid = lax.axis_index("c")
    sid = lax.axis_index("s")
    wid = sid * NC + cid
    sl = pl.ds(sid * S, S)

    pltpu.sync_copy(x_ref.at[sl, :], x_sh.at[sl, :])
    pltpu.sync_copy(zeros4_ref.at[sl, :], ao_sh.at[sl, :])
    pltpu.sync_copy(zeros4_ref.at[sl, :], ai_sh.at[sl, :])

    @pl.loop(0, S // SBLK)
    def _rdeg(b):
        base = sid * S + b * SBLK
        bsl = pl.ds(base, SBLK)
        pltpu.sync_copy(degop_ref.at[pl.ds(base, SBLK)], d0_v)
        pltpu.sync_copy(degop_ref.at[pl.ds(NP + base, SBLK)], d1_v)

        @pl.loop(0, SBLK // 16)
        def _rdo(i):
            ds16 = pl.ds(i * 16, 16)
            r_v[ds16] = 1.0 / (d0_v[ds16] + d1_v[ds16])

        pltpu.sync_copy(r_v, rdo_sh.at[bsl])
        pltpu.sync_copy(degip_ref.at[pl.ds(base, SBLK)], d0_v)
        pltpu.sync_copy(degip_ref.at[pl.ds(NP + base, SBLK)], d1_v)

        @pl.loop(0, SBLK // 16)
        def _rdi(i):
            ds16 = pl.ds(i * 16, 16)
            r_v[ds16] = 1.0 / (d0_v[ds16] + d1_v[ds16])

        pltpu.sync_copy(r_v, rdi_sh.at[bsl])

    plsc.subcore_barrier()

    lane = lax.iota(jnp.int32, 16)
    erep = lane >> 2
    crep = lane & 3

    @pl.loop(wid, NSUP, step=NW)
    def _edges(sc):
        pltpu.sync_copy(ei_ref.at[0, sc], sidx_v)
        pltpu.sync_copy(ei_ref.at[1, sc], didx_v)
        pltpu.sync_copy(w_ref.at[sc], w_v)
        for j in range(SUP):
            si = sidx_v.at[j]
            di = didx_v.at[j]
            pltpu.sync_copy(rdo_sh.at[si], rdo_b)
            pltpu.sync_copy(rdi_sh.at[di], rdi_b)
            pltpu.sync_copy(x_sh.at[si], xs_b)
            pltpu.sync_copy(x_sh.at[di], xd_b)

            @pl.loop(0, B // 16)
            def _norm(i):
                ds16 = pl.ds(i * 16, 16)
                wv = w_v[j, ds16]
                no_v[ds16] = wv * rdo_b[ds16]
                ni_v[ds16] = wv * rdi_b[ds16]

            @pl.loop(0, B * CIN // 16)
            def _mul(i):
                e16 = i * 4 + erep
                ng = plsc.load_gather(no_v, [e16])
                xg = plsc.load_gather(xs_b, [e16, crep])
                plsc.store_scatter(co_b, [e16, crep], xg * ng)
                ng2 = plsc.load_gather(ni_v, [e16])
                xg2 = plsc.load_gather(xd_b, [e16, crep])
                plsc.store_scatter(ci_b, [e16, crep], xg2 * ng2)

            pltpu.sync_copy(co_b, ao_sh.at[di], add=True)
            pltpu.sync_copy(ci_b, ai_sh.at[si], add=True)

    plsc.subcore_barrier()
    osl = pl.ds(cid * NP + sid * S, S)
    pltpu.sync_copy(ao_sh.at[sl, :], ao_out.at[osl, :])
    pltpu.sync_copy(ai_sh.at[sl, :], ai_out.at[osl, :])


BK = 2048


def _dense_body(xT_ref, aoT_ref, aiT_ref, wz_ref, bz_ref, wh_ref, bh_ref,
                wl_ref, bl_ref, out_ref):
    xb = xT_ref[...]
    ao = aoT_ref[0] + aoT_ref[1]
    ai = aiT_ref[0] + aiT_ref[1]
    f = jnp.concatenate([xb, ao, ai], axis=0)
    pz = jnp.dot(wz_ref[...], f, preferred_element_type=jnp.float32) + bz_ref[...]
    ph = jnp.dot(wh_ref[...], f, preferred_element_type=jnp.float32) + bh_ref[...]
    z = jax.nn.sigmoid(pz)
    ht = jnp.tanh(ph)
    g = jnp.maximum((1.0 - z) * ht, 0.0)
    out_ref[...] = jnp.dot(wl_ref[...], g, preferred_element_type=jnp.float32) + bl_ref[...]


_dense_call = pl.pallas_call(
    _dense_body,
    grid=(NP // BK,),
    in_specs=[
        pl.BlockSpec((4, BK), lambda i: (0, i)),
        pl.BlockSpec((NC, 4, BK), lambda i: (0, 0, i)),
        pl.BlockSpec((NC, 4, BK), lambda i: (0, 0, i)),
        pl.BlockSpec((32, 12), lambda i: (0, 0)),
        pl.BlockSpec((32, 1), lambda i: (0, 0)),
        pl.BlockSpec((32, 12), lambda i: (0, 0)),
        pl.BlockSpec((32, 1), lambda i: (0, 0)),
        pl.BlockSpec((1, 32), lambda i: (0, 0)),
        pl.BlockSpec((1, 1), lambda i: (0, 0)),
    ],
    out_specs=pl.BlockSpec((1, BK), lambda i: (0, i)),
    out_shape=jax.ShapeDtypeStruct((1, NP), jnp.float32),
)


def kernel(x, edge_index, edge_weight, W_z, b_z, W_r, b_r, W_h, b_h,
           W_lin, b_lin):
    del W_r, b_r
    ei2 = jnp.pad(edge_index, ((0, 0), (0, EP - N_EDGES)),
                  constant_values=N_NODES).reshape(2, NSUP, SUP, B)
    w2 = jnp.pad(edge_weight, (0, EP - N_EDGES),
                 constant_values=1.0).reshape(NSUP, SUP, B)
    x_pad = jnp.pad(x, ((0, NP - N_NODES), (0, 0)))
    zeros4 = jnp.zeros((NP, CIN), jnp.float32)

    dego_p, degi_p = _deg_kernel(ei2, w2)
    ao_p, ai_p = _scatter_kernel(ei2, w2, x_pad, dego_p, degi_p, zeros4)

    wz_f = jnp.concatenate(
        [W_z[0, 0, :CIN] + W_z[1, 0, :CIN], W_z[0, 1, :CIN], W_z[1, 1, :CIN]],
        axis=0).T
    wh_f = jnp.concatenate(
        [W_h[0, 0, :CIN] + W_h[1, 0, :CIN], W_h[0, 1, :CIN], W_h[1, 1, :CIN]],
        axis=0).T

    out_t = _dense_call(
        x_pad.T,
        jnp.transpose(ao_p.reshape(NC, NP, CIN), (0, 2, 1)),
        jnp.transpose(ai_p.reshape(NC, NP, CIN), (0, 2, 1)),
        wz_f, b_z[:, None], wh_f, b_h[:, None],
        W_lin.T, b_lin[:, None],
    )
    return out_t[0, :N_NODES][:, None]
